# baseline (speedup 1.0000x reference)
import math

import jax
import jax.numpy as jnp
from jax import lax
from jax.experimental import pallas as pl
from jax.experimental.pallas import tpu as pltpu

_B = 4096
_EMBED = 128
_MEM = 20
_NMEM = 100000
_NHEADS = 4
_ATT = 32
_NNEG = 5

_BB = 32
_NB = _B // _BB
_ROWS = _BB * _MEM
_NROWS = _BB * _NNEG
_PU = 8
_NU = _BB // _PU
_UROWS = _PU * _MEM


def _issue_gather(vm_ref, idx_ref, mem_scratch, sem, step, slot):
    for j in range(_BB):
        pltpu.make_async_copy(
            vm_ref.at[idx_ref[step * _BB + j]],
            mem_scratch.at[slot, pl.ds(_MEM * j, _MEM)],
            sem.at[slot],
        ).start()


def _wait_gather(vm_ref, idx_ref, mem_scratch, sem, step, slot):
    for j in range(_BB):
        pltpu.make_async_copy(
            vm_ref.at[idx_ref[step * _BB + j]],
            mem_scratch.at[slot, pl.ds(_MEM * j, _MEM)],
            sem.at[slot],
        ).wait()


def _body(idx_ref, vm_ref, attr_ref, users_ref, items_ref, negs_ref,
          key_ref, eW_ref, eb_ref, aW_ref, ab_ref,
          mW1_ref, mW2_ref, mb_ref, wq_ref, wk_ref, wv_ref,
          pos_ref, neg_ref, mem_scratch, sem):
    f32 = jnp.float32
    i = pl.program_id(0)
    slot = lax.rem(i, 2)

    @pl.when(i == 0)
    def _prologue():
        _issue_gather(vm_ref, idx_ref, mem_scratch, sem, 0, 0)

    @pl.when(i + 1 < _NB)
    def _prefetch():
        _issue_gather(vm_ref, idx_ref, mem_scratch, sem, i + 1,
                      lax.rem(i + 1, 2))

    attr = attr_ref[...]
    keym = key_ref[...]

    inner = jax.lax.dot_general(attr, keym, (((1,), (1,)), ((), ())))
    a_len = jnp.sqrt(jnp.sum(attr * attr, axis=1, keepdims=True))
    k_len = jnp.sqrt(jnp.sum(keym * keym, axis=1, keepdims=True))
    denom = jnp.dot(a_len, k_len.reshape(1, _MEM))
    cosine = inner / denom
    cmax = jnp.max(cosine, axis=1, keepdims=True)
    cexp = jnp.exp(cosine - cmax)
    cw = cexp / jnp.sum(cexp, axis=1, keepdims=True)

    erase_v = jax.nn.sigmoid(jnp.dot(attr, eW_ref[...]) + eb_ref[...])
    add_v = jnp.tanh(jnp.dot(attr, aW_ref[...]) + ab_ref[...])

    rb = lax.broadcasted_iota(jnp.int32, (_ROWS, _BB), 0) // _MEM
    cb = lax.broadcasted_iota(jnp.int32, (_ROWS, _BB), 1)
    S = jnp.where(rb == cb, f32(1.0), f32(0.0))

    erase_flat = jnp.dot(S, erase_v)
    add_flat = jnp.dot(S, add_v)
    cwb = jnp.dot(S, cw)
    ri = lax.broadcasted_iota(jnp.int32, (_ROWS, _MEM), 0) % _MEM
    li = lax.broadcasted_iota(jnp.int32, (_ROWS, _MEM), 1)
    cw_flat = jnp.sum(jnp.where(ri == li, cwb, f32(0.0)), axis=1,
                      keepdims=True)

    _wait_gather(vm_ref, idx_ref, mem_scratch, sem, i, slot)
    memf = mem_scratch[slot]

    x = memf * (1.0 - erase_flat * cw_flat) + add_flat * cw_flat

    qv = jnp.dot(x, wq_ref[...])
    kv = jnp.dot(x, wk_ref[...])
    vv = jnp.dot(x, wv_ref[...])

    r0 = lax.broadcasted_iota(jnp.int32, (_UROWS, _UROWS), 0) // _MEM
    c0 = lax.broadcasted_iota(jnp.int32, (_UROWS, _UROWS), 1) // _MEM
    blockmask = r0 == c0
    ar = lax.broadcasted_iota(jnp.int32, (_BB, _ROWS), 0)
    ac = lax.broadcasted_iota(jnp.int32, (_BB, _ROWS), 1) // _MEM
    A_avg = jnp.where(ar == ac, f32(1.0 / _MEM), f32(0.0))

    scale = f32(1.0 / math.sqrt(_EMBED))
    heads = []
    for h in range(_NHEADS):
        sl = slice(h * _ATT, (h + 1) * _ATT)
        qh = qv[:, sl]
        kh = kv[:, sl]
        vh = vv[:, sl]
        att_units = []
        for u in range(_NU):
            us = slice(u * _UROWS, (u + 1) * _UROWS)
            g = jax.lax.dot_general(qh[us], kh[us],
                                    (((1,), (1,)), ((), ()))) * scale
            g = jnp.where(blockmask, g, f32(-1e30))
            m = jnp.max(g, axis=1, keepdims=True)
            p = jnp.exp(g - m)
            s = jnp.sum(p, axis=1, keepdims=True)
            att_units.append(jnp.dot(p, vh[us]) / s)
        att = jnp.concatenate(att_units, axis=0)
        heads.append(jnp.dot(A_avg, att))
    memb = jnp.concatenate(heads, axis=1)

    norm = jnp.sqrt(jnp.sum(memb * memb, axis=1, keepdims=True))
    memb = memb / jnp.maximum(norm, f32(1e-12))

    merged = jnp.tanh(jnp.dot(users_ref[...], mW1_ref[...]) +
                      jnp.dot(memb, mW2_ref[...]) + mb_ref[...])

    pos_dot = jnp.sum(merged * items_ref[...], axis=1, keepdims=True)
    pos_part = jnp.sum(jnp.log(jax.nn.sigmoid(pos_dot) + 1e-24))

    nr = lax.broadcasted_iota(jnp.int32, (_NROWS, _BB), 0) // _NNEG
    nc = lax.broadcasted_iota(jnp.int32, (_NROWS, _BB), 1)
    S5 = jnp.where(nr == nc, f32(1.0), f32(0.0))
    mrows = jnp.dot(S5, merged)
    nd = jnp.sum(negs_ref[...] * mrows, axis=1, keepdims=True)
    one_minus = jnp.maximum(1.0 - jax.nn.sigmoid(nd), f32(0.0))
    neg_part = jnp.sum(jnp.log(one_minus + 1e-24))

    @pl.when(i == 0)
    def _init():
        pos_ref[0, 0] = f32(0.0)
        neg_ref[0, 0] = f32(0.0)

    pos_ref[0, 0] += pos_part
    neg_ref[0, 0] += neg_part


def _params(interpret=False):
    grid_spec = pltpu.PrefetchScalarGridSpec(
        num_scalar_prefetch=1,
        grid=(_NB,),
        in_specs=[
            pl.BlockSpec(memory_space=pl.ANY),
            pl.BlockSpec((_BB, _EMBED), lambda i, ir: (i, 0)),
            pl.BlockSpec((_BB, _EMBED), lambda i, ir: (i, 0)),
            pl.BlockSpec((_BB, _EMBED), lambda i, ir: (i, 0)),
            pl.BlockSpec((_NROWS, _EMBED), lambda i, ir: (i, 0)),
            pl.BlockSpec((_MEM, _EMBED), lambda i, ir: (0, 0)),
            pl.BlockSpec((_EMBED, _EMBED), lambda i, ir: (0, 0)),
            pl.BlockSpec((1, _EMBED), lambda i, ir: (0, 0)),
            pl.BlockSpec((_EMBED, _EMBED), lambda i, ir: (0, 0)),
            pl.BlockSpec((1, _EMBED), lambda i, ir: (0, 0)),
            pl.BlockSpec((_EMBED, _EMBED), lambda i, ir: (0, 0)),
            pl.BlockSpec((_EMBED, _EMBED), lambda i, ir: (0, 0)),
            pl.BlockSpec((1, _EMBED), lambda i, ir: (0, 0)),
            pl.BlockSpec((_EMBED, _EMBED), lambda i, ir: (0, 0)),
            pl.BlockSpec((_EMBED, _EMBED), lambda i, ir: (0, 0)),
            pl.BlockSpec((_EMBED, _EMBED), lambda i, ir: (0, 0)),
        ],
        out_specs=[
            pl.BlockSpec((1, 1), lambda i, ir: (0, 0), memory_space=pltpu.SMEM),
            pl.BlockSpec((1, 1), lambda i, ir: (0, 0), memory_space=pltpu.SMEM),
        ],
        scratch_shapes=[
            pltpu.VMEM((2, _ROWS, _EMBED), jnp.float32),
            pltpu.SemaphoreType.DMA((2,)),
        ],
    )
    out_shape = [
        jax.ShapeDtypeStruct((1, 1), jnp.float32),
        jax.ShapeDtypeStruct((1, 1), jnp.float32),
    ]
    return dict(grid_spec=grid_spec, out_shape=out_shape, interpret=interpret)


def kernel(idx, users_embed, items_embed, negs, delta_time, attr_vecs,
           value_matrix, key_matrix, erase_W, erase_b, add_W, add_b,
           merge_W, merge_b, att_key, att_query, att_value):
    del delta_time
    wq = att_query.transpose(1, 0, 2).reshape(_EMBED, _NHEADS * _ATT)
    wk = att_key.transpose(1, 0, 2).reshape(_EMBED, _NHEADS * _ATT)
    wv = att_value.transpose(1, 0, 2).reshape(_EMBED, _NHEADS * _ATT)
    negs_flat = negs.reshape(_B * _NNEG, _EMBED)

    pos_s, neg_s = pl.pallas_call(_body, **_params())(
        idx, value_matrix, attr_vecs, users_embed, items_embed, negs_flat,
        key_matrix, erase_W, erase_b.reshape(1, _EMBED),
        add_W, add_b.reshape(1, _EMBED),
        merge_W[:_EMBED], merge_W[_EMBED:], merge_b.reshape(1, _EMBED),
        wq, wk, wv)

    pos = pos_s[0, 0] / _B
    neg = neg_s[0, 0] / (_B * _NNEG)
    return -(pos + neg)


# speedup vs baseline: 1.0407x; 1.0407x over previous
import math

import jax
import jax.numpy as jnp
from jax import lax
from jax.experimental import pallas as pl
from jax.experimental.pallas import tpu as pltpu

_B = 4096
_EMBED = 128
_MEM = 20
_NMEM = 100000
_NHEADS = 4
_ATT = 32
_NNEG = 5

_BB = 32
_NB = _B // _BB
_ROWS = _BB * _MEM
_NROWS = _BB * _NNEG
_PU = 8
_NU = _BB // _PU
_UROWS = _PU * _MEM


def _issue_gather(vm_ref, idx_ref, mem_scratch, sem, step, slot):
    for j in range(_BB):
        pltpu.make_async_copy(
            vm_ref.at[idx_ref[step * _BB + j]],
            mem_scratch.at[slot, pl.ds(_MEM * j, _MEM)],
            sem.at[slot],
        ).start()


def _wait_gather(vm_ref, idx_ref, mem_scratch, sem, step, slot):
    for j in range(_BB):
        pltpu.make_async_copy(
            vm_ref.at[idx_ref[step * _BB + j]],
            mem_scratch.at[slot, pl.ds(_MEM * j, _MEM)],
            sem.at[slot],
        ).wait()


def _body(idx_ref, vm_ref, attr_ref, users_ref, items_ref, negs_ref,
          key_ref, eW_ref, eb_ref, aW_ref, ab_ref,
          mW_ref, mb_ref, wq_ref, wk_ref, wv_ref,
          pos_ref, neg_ref, mem_scratch, sem):
    f32 = jnp.float32
    i = pl.program_id(0)
    slot = lax.rem(i, 2)

    @pl.when(i == 0)
    def _prologue():
        _issue_gather(vm_ref, idx_ref, mem_scratch, sem, 0, 0)

    @pl.when(i + 1 < _NB)
    def _prefetch():
        _issue_gather(vm_ref, idx_ref, mem_scratch, sem, i + 1,
                      lax.rem(i + 1, 2))

    attr = attr_ref[...]
    keym = key_ref[...]

    inner = jax.lax.dot_general(attr, keym, (((1,), (1,)), ((), ())))
    a_len = jnp.sqrt(jnp.sum(attr * attr, axis=1, keepdims=True))
    k_len = jnp.sqrt(jnp.sum(keym * keym, axis=1, keepdims=True))
    denom = jnp.dot(a_len, k_len.reshape(1, _MEM))
    cosine = inner / denom
    cmax = jnp.max(cosine, axis=1, keepdims=True)
    cexp = jnp.exp(cosine - cmax)
    cw = cexp / jnp.sum(cexp, axis=1, keepdims=True)

    erase_v = jax.nn.sigmoid(jnp.dot(attr, eW_ref[...]) + eb_ref[...])
    add_v = jnp.tanh(jnp.dot(attr, aW_ref[...]) + ab_ref[...])

    rb = lax.broadcasted_iota(jnp.int32, (_ROWS, _BB), 0) // _MEM
    cb = lax.broadcasted_iota(jnp.int32, (_ROWS, _BB), 1)
    S = jnp.where(rb == cb, f32(1.0), f32(0.0))

    erase_flat = jnp.dot(S, erase_v)
    add_flat = jnp.dot(S, add_v)
    cwb = jnp.dot(S, cw)
    ri = lax.broadcasted_iota(jnp.int32, (_ROWS, _MEM), 0) % _MEM
    li = lax.broadcasted_iota(jnp.int32, (_ROWS, _MEM), 1)
    cw_flat = jnp.sum(jnp.where(ri == li, cwb, f32(0.0)), axis=1,
                      keepdims=True)

    _wait_gather(vm_ref, idx_ref, mem_scratch, sem, i, slot)
    memf = mem_scratch[slot]

    x = memf * (1.0 - erase_flat * cw_flat) + add_flat * cw_flat

    r0 = lax.broadcasted_iota(jnp.int32, (_UROWS, _UROWS), 0) // _MEM
    c0 = lax.broadcasted_iota(jnp.int32, (_UROWS, _UROWS), 1) // _MEM
    blockmask = r0 == c0
    ar = lax.broadcasted_iota(jnp.int32, (_BB, _ROWS), 0)
    ac = lax.broadcasted_iota(jnp.int32, (_BB, _ROWS), 1) // _MEM
    A_avg = jnp.where(ar == ac, f32(1.0 / _MEM), f32(0.0))

    scale = f32(1.0 / math.sqrt(_EMBED))
    heads = []
    for h in range(_NHEADS):
        qh = jnp.dot(x, wq_ref[h])
        kh = jnp.dot(x, wk_ref[h])
        vh = jnp.dot(x, wv_ref[h])
        att_units = []
        for u in range(_NU):
            us = slice(u * _UROWS, (u + 1) * _UROWS)
            g = jax.lax.dot_general(qh[us], kh[us],
                                    (((1,), (1,)), ((), ()))) * scale
            g = jnp.where(blockmask, g, f32(-1e30))
            m = jnp.max(g, axis=1, keepdims=True)
            p = jnp.exp(g - m)
            s = jnp.sum(p, axis=1, keepdims=True)
            att_units.append(jnp.dot(p, vh[us]) / s)
        att = jnp.concatenate(att_units, axis=0)
        heads.append(jnp.dot(A_avg, att))
    memb = jnp.concatenate(heads, axis=1)

    norm = jnp.sqrt(jnp.sum(memb * memb, axis=1, keepdims=True))
    memb = memb / jnp.maximum(norm, f32(1e-12))

    mW = mW_ref[...]
    merged = jnp.tanh(jnp.dot(users_ref[...], mW[:_EMBED]) +
                      jnp.dot(memb, mW[_EMBED:]) + mb_ref[...])

    pos_dot = jnp.sum(merged * items_ref[...], axis=1, keepdims=True)
    pos_part = jnp.sum(jnp.log(jax.nn.sigmoid(pos_dot) + 1e-24))

    nd = jnp.sum(negs_ref[...] * merged[:, None, :], axis=2)
    one_minus = jnp.maximum(1.0 - jax.nn.sigmoid(nd), f32(0.0))
    neg_part = jnp.sum(jnp.log(one_minus + 1e-24))

    @pl.when(i == 0)
    def _init():
        pos_ref[0, 0] = f32(0.0)
        neg_ref[0, 0] = f32(0.0)

    pos_ref[0, 0] += pos_part
    neg_ref[0, 0] += neg_part


def _params(interpret=False):
    grid_spec = pltpu.PrefetchScalarGridSpec(
        num_scalar_prefetch=1,
        grid=(_NB,),
        in_specs=[
            pl.BlockSpec(memory_space=pl.ANY),
            pl.BlockSpec((_BB, _EMBED), lambda i, ir: (i, 0)),
            pl.BlockSpec((_BB, _EMBED), lambda i, ir: (i, 0)),
            pl.BlockSpec((_BB, _EMBED), lambda i, ir: (i, 0)),
            pl.BlockSpec((_BB, _NNEG, _EMBED), lambda i, ir: (i, 0, 0)),
            pl.BlockSpec((_MEM, _EMBED), lambda i, ir: (0, 0)),
            pl.BlockSpec((_EMBED, _EMBED), lambda i, ir: (0, 0)),
            pl.BlockSpec((1, _EMBED), lambda i, ir: (0, 0)),
            pl.BlockSpec((_EMBED, _EMBED), lambda i, ir: (0, 0)),
            pl.BlockSpec((1, _EMBED), lambda i, ir: (0, 0)),
            pl.BlockSpec((2 * _EMBED, _EMBED), lambda i, ir: (0, 0)),
            pl.BlockSpec((1, _EMBED), lambda i, ir: (0, 0)),
            pl.BlockSpec((_NHEADS, _EMBED, _ATT), lambda i, ir: (0, 0, 0)),
            pl.BlockSpec((_NHEADS, _EMBED, _ATT), lambda i, ir: (0, 0, 0)),
            pl.BlockSpec((_NHEADS, _EMBED, _ATT), lambda i, ir: (0, 0, 0)),
        ],
        out_specs=[
            pl.BlockSpec((1, 1), lambda i, ir: (0, 0), memory_space=pltpu.SMEM),
            pl.BlockSpec((1, 1), lambda i, ir: (0, 0), memory_space=pltpu.SMEM),
        ],
        scratch_shapes=[
            pltpu.VMEM((2, _ROWS, _EMBED), jnp.float32),
            pltpu.SemaphoreType.DMA((2,)),
        ],
    )
    out_shape = [
        jax.ShapeDtypeStruct((1, 1), jnp.float32),
        jax.ShapeDtypeStruct((1, 1), jnp.float32),
    ]
    return dict(grid_spec=grid_spec, out_shape=out_shape, interpret=interpret)


def kernel(idx, users_embed, items_embed, negs, delta_time, attr_vecs,
           value_matrix, key_matrix, erase_W, erase_b, add_W, add_b,
           merge_W, merge_b, att_key, att_query, att_value):
    del delta_time
    pos_s, neg_s = pl.pallas_call(_body, **_params())(
        idx, value_matrix, attr_vecs, users_embed, items_embed, negs,
        key_matrix, erase_W, erase_b.reshape(1, _EMBED),
        add_W, add_b.reshape(1, _EMBED),
        merge_W, merge_b.reshape(1, _EMBED),
        att_query, att_key, att_value)

    pos = pos_s[0, 0] / _B
    neg = neg_s[0, 0] / (_B * _NNEG)
    return -(pos + neg)
